# Initial kernel scaffold; baseline (speedup 1.0000x reference)
import jax
import jax.numpy as jnp
from jax.experimental import pallas as pl


def kernel(encoder_output, his_turn_end_ids, W_ih, W_hh, b_ih, b_hh, W_fc, b_fc):
    raise NotImplementedError("write your pallas kernel here")


# speedup vs baseline: 5.6230x; 5.6230x over previous
import jax
import jax.numpy as jnp
from jax import lax
from jax.experimental import pallas as pl
from jax.experimental.pallas import tpu as pltpu

BSZ, SEQ, ENC = 8, 2048, 512
HID = 512
T = 16
SBLK = 512
NS = SEQ // SBLK


def _seg_mean_body(x_ref, ends_ref, prev_ref, invc_ref, wfc_ref,
                   states_ref, bs_ref):
    s = pl.program_id(1)
    x = x_ref[0]
    ends = ends_ref[0]
    prev = prev_ref[0]
    invc = invc_ref[0]
    pos = lax.broadcasted_iota(jnp.int32, (SBLK, T), 0) + s * SBLK
    m = jnp.where((pos > prev) & (pos <= ends), invc, 0.0)
    contrib = lax.dot_general(m, x, (((0,), (0,)), ((), ())),
                              preferred_element_type=jnp.float32)
    ws = wfc_ref[0:1, HID:HID + ENC]
    y = lax.dot_general(x, ws, (((1,), (1,)), ((), ())),
                        preferred_element_type=jnp.float32)
    bs_c = lax.dot_general(y, m, (((0,), (0,)), ((), ())),
                           preferred_element_type=jnp.float32)

    @pl.when(s == 0)
    def _():
        states_ref[0] = contrib
        bs_ref[0] = bs_c

    @pl.when(s != 0)
    def _():
        states_ref[0] += contrib
        bs_ref[0] += bs_c


def _lstm_loss_body(states_ref, bs_ref, wih_ref, whh_ref, bias_ref,
                    wfc_ref, bfc_ref, out_ref):
    states = states_ref[...]
    sf = states.reshape(BSZ * T, ENC)
    bias = bias_ref[...]
    G = lax.dot_general(sf, wih_ref[...], (((1,), (1,)), ((), ())),
                        preferred_element_type=jnp.float32) + bias
    G3 = G.reshape(BSZ, T, 4 * HID)

    h = jnp.zeros((BSZ * T, HID), jnp.float32)
    c = jnp.zeros((BSZ * T, HID), jnp.float32)
    for t in range(3):
        if t == 0:
            xg = G
        else:
            xg = jnp.concatenate([G3[:, t:, :], G3[:, :t, :]],
                                 axis=1).reshape(BSZ * T, 4 * HID)
        gates = xg + lax.dot_general(h, whh_ref[...], (((1,), (1,)), ((), ())),
                                     preferred_element_type=jnp.float32)
        i_g = jax.nn.sigmoid(gates[:, 0:HID])
        f_g = jax.nn.sigmoid(gates[:, HID:2 * HID])
        g_g = jnp.tanh(gates[:, 2 * HID:3 * HID])
        o_g = jax.nn.sigmoid(gates[:, 3 * HID:4 * HID])
        c = f_g * c + i_g * g_g
        h = o_g * jnp.tanh(c)

    wh = wfc_ref[0:1, 0:HID]
    a = jnp.sum(h * wh, axis=1, keepdims=True) + bfc_ref[0, 0]
    a3 = a.reshape(BSZ, T, 1)
    bs3 = bs_ref[...]
    logits = a3 + bs3
    j_idx = lax.broadcasted_iota(jnp.int32, (BSZ, T, T), 1)
    k_idx = lax.broadcasted_iota(jnp.int32, (BSZ, T, T), 2)
    valid = k_idx >= (j_idx + 3)
    lm = jnp.where(valid, logits, -1e30)
    mx = jnp.max(lm, axis=2, keepdims=True)
    logz = mx + jnp.log(jnp.sum(jnp.exp(lm - mx), axis=2, keepdims=True))
    tgt = jnp.sum(jnp.where(k_idx == j_idx + 3, logits, 0.0),
                  axis=2, keepdims=True)
    val = logz - tgt
    jmask = lax.broadcasted_iota(jnp.int32, (BSZ, T, 1), 1) < (T - 3)
    loss = jnp.sum(jnp.where(jmask, val, 0.0)) / (BSZ * (T - 3))
    out_ref[0, 0] = loss


def kernel(encoder_output, his_turn_end_ids, W_ih, W_hh, b_ih, b_hh,
           W_fc, b_fc):
    ends = his_turn_end_ids.astype(jnp.int32)
    prev = jnp.concatenate(
        [jnp.full((BSZ, 1), -1, jnp.int32), ends[:, :-1]], axis=1)
    invc = 1.0 / (ends - prev).astype(jnp.float32)
    ends3 = ends.reshape(BSZ, 1, T)
    prev3 = prev.reshape(BSZ, 1, T)
    invc3 = invc.reshape(BSZ, 1, T)
    bias = (b_ih + b_hh).reshape(1, 4 * HID)
    bfc = b_fc.reshape(1, 1)

    states, bs = pl.pallas_call(
        _seg_mean_body,
        grid=(BSZ, NS),
        in_specs=[
            pl.BlockSpec((1, SBLK, ENC), lambda b, s: (b, s, 0)),
            pl.BlockSpec((1, 1, T), lambda b, s: (b, 0, 0)),
            pl.BlockSpec((1, 1, T), lambda b, s: (b, 0, 0)),
            pl.BlockSpec((1, 1, T), lambda b, s: (b, 0, 0)),
            pl.BlockSpec((1, HID + ENC), lambda b, s: (0, 0)),
        ],
        out_specs=[
            pl.BlockSpec((1, T, ENC), lambda b, s: (b, 0, 0)),
            pl.BlockSpec((1, 1, T), lambda b, s: (b, 0, 0)),
        ],
        out_shape=[
            jax.ShapeDtypeStruct((BSZ, T, ENC), jnp.float32),
            jax.ShapeDtypeStruct((BSZ, 1, T), jnp.float32),
        ],
    )(encoder_output, ends3, prev3, invc3, W_fc)

    loss2d = pl.pallas_call(
        _lstm_loss_body,
        in_specs=[
            pl.BlockSpec(memory_space=pltpu.VMEM),
            pl.BlockSpec(memory_space=pltpu.VMEM),
            pl.BlockSpec(memory_space=pltpu.VMEM),
            pl.BlockSpec(memory_space=pltpu.VMEM),
            pl.BlockSpec(memory_space=pltpu.VMEM),
            pl.BlockSpec(memory_space=pltpu.VMEM),
            pl.BlockSpec(memory_space=pltpu.SMEM),
        ],
        out_specs=pl.BlockSpec(memory_space=pltpu.SMEM),
        out_shape=jax.ShapeDtypeStruct((1, 1), jnp.float32),
    )(states, bs, W_ih, W_hh, bias, W_fc, bfc)
    return loss2d[0, 0]


# speedup vs baseline: 7.7242x; 1.3737x over previous
import jax
import jax.numpy as jnp
from jax import lax
from jax.experimental import pallas as pl
from jax.experimental.pallas import tpu as pltpu

BSZ, SEQ, ENC = 8, 2048, 512
HID = 512
T = 16
SBLK = 1024
NS = SEQ // SBLK


def _seg_mean_body(x_ref, ends_ref, prev_ref, invc_ref, states_ref):
    s = pl.program_id(1)
    x = x_ref[0]
    ends = ends_ref[0]
    prev = prev_ref[0]
    invc = invc_ref[0]
    pos = lax.broadcasted_iota(jnp.int32, (SBLK, T), 0) + s * SBLK
    m = jnp.where((pos > prev) & (pos <= ends), invc, 0.0)
    contrib = lax.dot_general(m, x, (((0,), (0,)), ((), ())),
                              preferred_element_type=jnp.float32)

    @pl.when(s == 0)
    def _():
        states_ref[0] = contrib

    @pl.when(s != 0)
    def _():
        states_ref[0] += contrib


def _lstm_loss_body(states_ref, wih_ref, whh_ref, bias_ref,
                    wfc_ref, bfc_ref, out_ref):
    states = states_ref[...]
    sf = states.reshape(BSZ * T, ENC)
    ws = wfc_ref[0:1, HID:HID + ENC]
    bs3 = jnp.concatenate(
        [lax.dot_general(ws, states[b], (((1,), (1,)), ((), ())),
                         preferred_element_type=jnp.float32).reshape(1, 1, T)
         for b in range(BSZ)], axis=0)
    bias = bias_ref[...]
    G = lax.dot_general(sf, wih_ref[...], (((1,), (1,)), ((), ())),
                        preferred_element_type=jnp.float32) + bias
    G3 = G.reshape(BSZ, T, 4 * HID)

    h = jnp.zeros((BSZ * T, HID), jnp.float32)
    c = jnp.zeros((BSZ * T, HID), jnp.float32)
    for t in range(3):
        if t == 0:
            xg = G
        else:
            xg = jnp.concatenate([G3[:, t:, :], G3[:, :t, :]],
                                 axis=1).reshape(BSZ * T, 4 * HID)
        gates = xg + lax.dot_general(h, whh_ref[...], (((1,), (1,)), ((), ())),
                                     preferred_element_type=jnp.float32)
        i_g = jax.nn.sigmoid(gates[:, 0:HID])
        f_g = jax.nn.sigmoid(gates[:, HID:2 * HID])
        g_g = jnp.tanh(gates[:, 2 * HID:3 * HID])
        o_g = jax.nn.sigmoid(gates[:, 3 * HID:4 * HID])
        c = f_g * c + i_g * g_g
        h = o_g * jnp.tanh(c)

    wh = wfc_ref[0:1, 0:HID]
    a = jnp.sum(h * wh, axis=1, keepdims=True) + bfc_ref[0, 0]
    a3 = a.reshape(BSZ, T, 1)
    logits = a3 + bs3
    j_idx = lax.broadcasted_iota(jnp.int32, (BSZ, T, T), 1)
    k_idx = lax.broadcasted_iota(jnp.int32, (BSZ, T, T), 2)
    valid = k_idx >= (j_idx + 3)
    lm = jnp.where(valid, logits, -1e30)
    mx = jnp.max(lm, axis=2, keepdims=True)
    logz = mx + jnp.log(jnp.sum(jnp.exp(lm - mx), axis=2, keepdims=True))
    tgt = jnp.sum(jnp.where(k_idx == j_idx + 3, logits, 0.0),
                  axis=2, keepdims=True)
    val = logz - tgt
    jmask = lax.broadcasted_iota(jnp.int32, (BSZ, T, 1), 1) < (T - 3)
    loss = jnp.sum(jnp.where(jmask, val, 0.0)) / (BSZ * (T - 3))
    out_ref[0, 0] = loss


def kernel(encoder_output, his_turn_end_ids, W_ih, W_hh, b_ih, b_hh,
           W_fc, b_fc):
    ends = his_turn_end_ids.astype(jnp.int32)
    prev = jnp.concatenate(
        [jnp.full((BSZ, 1), -1, jnp.int32), ends[:, :-1]], axis=1)
    invc = 1.0 / (ends - prev).astype(jnp.float32)
    ends3 = ends.reshape(BSZ, 1, T)
    prev3 = prev.reshape(BSZ, 1, T)
    invc3 = invc.reshape(BSZ, 1, T)
    bias = (b_ih + b_hh).reshape(1, 4 * HID)
    bfc = b_fc.reshape(1, 1)

    states = pl.pallas_call(
        _seg_mean_body,
        grid=(BSZ, NS),
        in_specs=[
            pl.BlockSpec((1, SBLK, ENC), lambda b, s: (b, s, 0)),
            pl.BlockSpec((1, 1, T), lambda b, s: (b, 0, 0)),
            pl.BlockSpec((1, 1, T), lambda b, s: (b, 0, 0)),
            pl.BlockSpec((1, 1, T), lambda b, s: (b, 0, 0)),
        ],
        out_specs=pl.BlockSpec((1, T, ENC), lambda b, s: (b, 0, 0)),
        out_shape=jax.ShapeDtypeStruct((BSZ, T, ENC), jnp.float32),
    )(encoder_output, ends3, prev3, invc3)

    loss2d = pl.pallas_call(
        _lstm_loss_body,
        in_specs=[
            pl.BlockSpec(memory_space=pltpu.VMEM),
            pl.BlockSpec(memory_space=pltpu.VMEM),
            pl.BlockSpec(memory_space=pltpu.VMEM),
            pl.BlockSpec(memory_space=pltpu.VMEM),
            pl.BlockSpec(memory_space=pltpu.VMEM),
            pl.BlockSpec(memory_space=pltpu.SMEM),
        ],
        out_specs=pl.BlockSpec(memory_space=pltpu.SMEM),
        out_shape=jax.ShapeDtypeStruct((1, 1), jnp.float32),
    )(states, W_ih, W_hh, bias, W_fc, bfc)
    return loss2d[0, 0]


# speedup vs baseline: 9.0535x; 1.1721x over previous
import jax
import jax.numpy as jnp
from jax import lax
from jax.experimental import pallas as pl
from jax.experimental.pallas import tpu as pltpu

BSZ, SEQ, ENC = 8, 2048, 512
HID = 512
T = 16
SBLK = 2048
NS = SEQ // SBLK


def _seg_mean_body(x_ref, ends_ref, prev_ref, invc_ref, states_ref):
    s = pl.program_id(1)
    x = x_ref[0]
    ends = ends_ref[0]
    prev = prev_ref[0]
    invc = invc_ref[0]
    pos = lax.broadcasted_iota(jnp.int32, (SBLK, T), 0) + s * SBLK
    m = jnp.where((pos > prev) & (pos <= ends), invc, 0.0)
    contrib = lax.dot_general(m, x, (((0,), (0,)), ((), ())),
                              preferred_element_type=jnp.float32)

    @pl.when(s == 0)
    def _():
        states_ref[0] = contrib

    @pl.when(s != 0)
    def _():
        states_ref[0] += contrib


def _lstm_loss_body(states_ref, wih_ref, whh_ref, bias_ref,
                    wfc_ref, bfc_ref, out_ref):
    states = states_ref[...]
    sf = states.reshape(BSZ * T, ENC)
    ws = wfc_ref[0:1, HID:HID + ENC]
    bs3 = jnp.concatenate(
        [lax.dot_general(ws, states[b], (((1,), (1,)), ((), ())),
                         preferred_element_type=jnp.float32).reshape(1, 1, T)
         for b in range(BSZ)], axis=0)
    bias = bias_ref[...]
    G = lax.dot_general(sf, wih_ref[...], (((1,), (1,)), ((), ())),
                        preferred_element_type=jnp.float32) + bias
    G3 = G.reshape(BSZ, T, 4 * HID)

    h = jnp.zeros((BSZ * T, HID), jnp.float32)
    c = jnp.zeros((BSZ * T, HID), jnp.float32)
    for t in range(3):
        if t == 0:
            xg = G
        else:
            xg = jnp.concatenate([G3[:, t:, :], G3[:, :t, :]],
                                 axis=1).reshape(BSZ * T, 4 * HID)
        gates = xg + lax.dot_general(h, whh_ref[...], (((1,), (1,)), ((), ())),
                                     preferred_element_type=jnp.float32)
        i_g = jax.nn.sigmoid(gates[:, 0:HID])
        f_g = jax.nn.sigmoid(gates[:, HID:2 * HID])
        g_g = jnp.tanh(gates[:, 2 * HID:3 * HID])
        o_g = jax.nn.sigmoid(gates[:, 3 * HID:4 * HID])
        c = f_g * c + i_g * g_g
        h = o_g * jnp.tanh(c)

    wh = wfc_ref[0:1, 0:HID]
    a = jnp.sum(h * wh, axis=1, keepdims=True) + bfc_ref[0, 0]
    a3 = a.reshape(BSZ, T, 1)
    logits = a3 + bs3
    j_idx = lax.broadcasted_iota(jnp.int32, (BSZ, T, T), 1)
    k_idx = lax.broadcasted_iota(jnp.int32, (BSZ, T, T), 2)
    valid = k_idx >= (j_idx + 3)
    lm = jnp.where(valid, logits, -1e30)
    mx = jnp.max(lm, axis=2, keepdims=True)
    logz = mx + jnp.log(jnp.sum(jnp.exp(lm - mx), axis=2, keepdims=True))
    tgt = jnp.sum(jnp.where(k_idx == j_idx + 3, logits, 0.0),
                  axis=2, keepdims=True)
    val = logz - tgt
    jmask = lax.broadcasted_iota(jnp.int32, (BSZ, T, 1), 1) < (T - 3)
    loss = jnp.sum(jnp.where(jmask, val, 0.0)) / (BSZ * (T - 3))
    out_ref[0, 0] = loss


def kernel(encoder_output, his_turn_end_ids, W_ih, W_hh, b_ih, b_hh,
           W_fc, b_fc):
    ends = his_turn_end_ids.astype(jnp.int32)
    prev = jnp.concatenate(
        [jnp.full((BSZ, 1), -1, jnp.int32), ends[:, :-1]], axis=1)
    invc = 1.0 / (ends - prev).astype(jnp.float32)
    ends3 = ends.reshape(BSZ, 1, T)
    prev3 = prev.reshape(BSZ, 1, T)
    invc3 = invc.reshape(BSZ, 1, T)
    bias = (b_ih + b_hh).reshape(1, 4 * HID)
    bfc = b_fc.reshape(1, 1)

    states = pl.pallas_call(
        _seg_mean_body,
        grid=(BSZ, NS),
        in_specs=[
            pl.BlockSpec((1, SBLK, ENC), lambda b, s: (b, s, 0)),
            pl.BlockSpec((1, 1, T), lambda b, s: (b, 0, 0)),
            pl.BlockSpec((1, 1, T), lambda b, s: (b, 0, 0)),
            pl.BlockSpec((1, 1, T), lambda b, s: (b, 0, 0)),
        ],
        out_specs=pl.BlockSpec((1, T, ENC), lambda b, s: (b, 0, 0)),
        out_shape=jax.ShapeDtypeStruct((BSZ, T, ENC), jnp.float32),
    )(encoder_output, ends3, prev3, invc3)

    loss2d = pl.pallas_call(
        _lstm_loss_body,
        in_specs=[
            pl.BlockSpec(memory_space=pltpu.VMEM),
            pl.BlockSpec(memory_space=pltpu.VMEM),
            pl.BlockSpec(memory_space=pltpu.VMEM),
            pl.BlockSpec(memory_space=pltpu.VMEM),
            pl.BlockSpec(memory_space=pltpu.VMEM),
            pl.BlockSpec(memory_space=pltpu.SMEM),
        ],
        out_specs=pl.BlockSpec(memory_space=pltpu.SMEM),
        out_shape=jax.ShapeDtypeStruct((1, 1), jnp.float32),
    )(states, W_ih, W_hh, bias, W_fc, bfc)
    return loss2d[0, 0]


# speedup vs baseline: 9.4268x; 1.0412x over previous
import jax
import jax.numpy as jnp
from jax import lax
from jax.experimental import pallas as pl
from jax.experimental.pallas import tpu as pltpu

BSZ, SEQ, ENC = 8, 2048, 512
HID = 512
T = 16


def _fused_body(x_ref, ends_ref, prev_ref, invc_ref, wih_ref, whh_ref,
                bias_ref, wfc_ref, bfc_ref, out_ref, st_ref):
    bi = pl.program_id(0)
    x = x_ref[0]
    ends = ends_ref[0]
    prev = prev_ref[0]
    invc = invc_ref[0]
    pos = lax.broadcasted_iota(jnp.int32, (SEQ, T), 0)
    m = jnp.where((pos > prev) & (pos <= ends), invc, 0.0)
    st_ref[pl.ds(bi * T, T), :] = lax.dot_general(
        m, x, (((0,), (0,)), ((), ())),
        preferred_element_type=jnp.float32)

    @pl.when(bi == BSZ - 1)
    def _tail():
        sf = st_ref[...]
        states = sf.reshape(BSZ, T, ENC)
        ws = wfc_ref[0:1, HID:HID + ENC]
        bs3 = jnp.concatenate(
            [lax.dot_general(ws, states[b], (((1,), (1,)), ((), ())),
                             preferred_element_type=jnp.float32
                             ).reshape(1, 1, T)
             for b in range(BSZ)], axis=0)
        bias = bias_ref[...]
        G = lax.dot_general(sf, wih_ref[...], (((1,), (1,)), ((), ())),
                            preferred_element_type=jnp.float32) + bias
        G3 = G.reshape(BSZ, T, 4 * HID)

        h = jnp.zeros((BSZ * T, HID), jnp.float32)
        c = jnp.zeros((BSZ * T, HID), jnp.float32)
        for t in range(3):
            if t == 0:
                xg = G
            else:
                xg = jnp.concatenate([G3[:, t:, :], G3[:, :t, :]],
                                     axis=1).reshape(BSZ * T, 4 * HID)
            gates = xg + lax.dot_general(
                h, whh_ref[...], (((1,), (1,)), ((), ())),
                preferred_element_type=jnp.float32)
            i_g = jax.nn.sigmoid(gates[:, 0:HID])
            f_g = jax.nn.sigmoid(gates[:, HID:2 * HID])
            g_g = jnp.tanh(gates[:, 2 * HID:3 * HID])
            o_g = jax.nn.sigmoid(gates[:, 3 * HID:4 * HID])
            c = f_g * c + i_g * g_g
            h = o_g * jnp.tanh(c)

        wh = wfc_ref[0:1, 0:HID]
        a = jnp.sum(h * wh, axis=1, keepdims=True) + bfc_ref[0, 0]
        a3 = a.reshape(BSZ, T, 1)
        logits = a3 + bs3
        j_idx = lax.broadcasted_iota(jnp.int32, (BSZ, T, T), 1)
        k_idx = lax.broadcasted_iota(jnp.int32, (BSZ, T, T), 2)
        valid = k_idx >= (j_idx + 3)
        lm = jnp.where(valid, logits, -1e30)
        mx = jnp.max(lm, axis=2, keepdims=True)
        logz = mx + jnp.log(jnp.sum(jnp.exp(lm - mx), axis=2, keepdims=True))
        tgt = jnp.sum(jnp.where(k_idx == j_idx + 3, logits, 0.0),
                      axis=2, keepdims=True)
        val = logz - tgt
        jmask = lax.broadcasted_iota(jnp.int32, (BSZ, T, 1), 1) < (T - 3)
        out_ref[0, 0] = jnp.sum(jnp.where(jmask, val, 0.0)) / (BSZ * (T - 3))


def kernel(encoder_output, his_turn_end_ids, W_ih, W_hh, b_ih, b_hh,
           W_fc, b_fc):
    ends = his_turn_end_ids.astype(jnp.int32)
    prev = jnp.concatenate(
        [jnp.full((BSZ, 1), -1, jnp.int32), ends[:, :-1]], axis=1)
    invc = 1.0 / (ends - prev).astype(jnp.float32)
    ends3 = ends.reshape(BSZ, 1, T)
    prev3 = prev.reshape(BSZ, 1, T)
    invc3 = invc.reshape(BSZ, 1, T)
    bias = (b_ih + b_hh).reshape(1, 4 * HID)
    bfc = b_fc.reshape(1, 1)

    loss2d = pl.pallas_call(
        _fused_body,
        grid=(BSZ,),
        in_specs=[
            pl.BlockSpec((1, SEQ, ENC), lambda b: (b, 0, 0)),
            pl.BlockSpec((1, 1, T), lambda b: (b, 0, 0)),
            pl.BlockSpec((1, 1, T), lambda b: (b, 0, 0)),
            pl.BlockSpec((1, 1, T), lambda b: (b, 0, 0)),
            pl.BlockSpec((4 * HID, ENC), lambda b: (0, 0)),
            pl.BlockSpec((4 * HID, HID), lambda b: (0, 0)),
            pl.BlockSpec((1, 4 * HID), lambda b: (0, 0)),
            pl.BlockSpec((1, HID + ENC), lambda b: (0, 0)),
            pl.BlockSpec(memory_space=pltpu.SMEM),
        ],
        out_specs=pl.BlockSpec(memory_space=pltpu.SMEM),
        out_shape=jax.ShapeDtypeStruct((1, 1), jnp.float32),
        scratch_shapes=[pltpu.VMEM((BSZ * T, ENC), jnp.float32)],
    )(encoder_output, ends3, prev3, invc3, W_ih, W_hh, bias, W_fc, bfc)
    return loss2d[0, 0]
